# baseline (speedup 1.0000x reference)
import functools

import jax
import jax.numpy as jnp
from jax import lax
from jax.experimental import pallas as pl
from jax.experimental.pallas import tpu as pltpu
from jax.experimental.pallas import tpu_sc as plsc

_NUM_CORES = 2
_NUM_SUBCORES = 16
_LANES = 16

_M = 4
_B = 32
_N = 8192
_CHUNK = 4096
_NCHUNKS = _N // _CHUNK
_STEPS = _CHUNK // _LANES


def _se3_body(xyz_hbm, coef_hbm, out_hbm, in_v, out_v, coef_v,
              coef_sem, in_sems, out_sems):
    c = lax.axis_index("c")
    s = lax.axis_index("s")
    w = s * _NUM_CORES + c

    for m in range(_M):
        pltpu.make_async_copy(
            coef_hbm.at[m * _B + w], coef_v[m], coef_sem).start()
    for m in range(_M):
        pltpu.make_async_copy(
            coef_hbm.at[m * _B + w], coef_v[m], coef_sem).wait()
    rows = [coef_v[m][...] for m in range(_M)]
    cf = [[rows[m][j] for j in (0, 1, 2, 4, 5, 6, 8, 9, 10, 3, 7, 11)]
          for m in range(_M)]

    def in_copy(k, d):
        slot = k % 2
        return pltpu.make_async_copy(
            xyz_hbm.at[d, w, pl.ds(k * _CHUNK, _CHUNK)],
            in_v[slot][d], in_sems[slot])

    def out_copy(k, m, d):
        slot = k % 2
        return pltpu.make_async_copy(
            out_v[slot][3 * m + d],
            out_hbm.at[d, m * _B + w, pl.ds(k * _CHUNK, _CHUNK)],
            out_sems[slot])

    def compute(slot):
        xs, ys, zs = in_v[slot]
        dsts = out_v[slot]

        @plsc.parallel_loop(0, _STEPS, unroll=4)
        def body(i):
            o = i * _LANES
            sl = pl.ds(o, _LANES)
            x = xs[sl]
            y = ys[sl]
            z = zs[sl]
            for m in range(_M):
                r = cf[m]
                dsts[3 * m][sl] = x * r[0] + y * r[1] + z * r[2] + r[9]
                dsts[3 * m + 1][sl] = x * r[3] + y * r[4] + z * r[5] + r[10]
                dsts[3 * m + 2][sl] = x * r[6] + y * r[7] + z * r[8] + r[11]

    for d in range(3):
        in_copy(0, d).start()
    for k in range(_NCHUNKS):
        slot = k % 2
        if k + 1 < _NCHUNKS:
            for d in range(3):
                in_copy(k + 1, d).start()
        for d in range(3):
            in_copy(k, d).wait()
        if k >= 2:
            for m in range(_M):
                for d in range(3):
                    out_copy(k - 2, m, d).wait()
        compute(slot)
        for m in range(_M):
            for d in range(3):
                out_copy(k, m, d).start()
    for k in (_NCHUNKS - 2, _NCHUNKS - 1):
        for m in range(_M):
            for d in range(3):
                out_copy(k, m, d).wait()


@jax.jit
def kernel(trans, pos):
    m, b = trans.shape[0], trans.shape[1]
    n = pos.shape[1]
    coef = trans.reshape(m * b, 16)
    xyz = jnp.transpose(pos, (2, 0, 1))

    mesh = plsc.VectorSubcoreMesh(
        core_axis_name="c", subcore_axis_name="s",
        num_cores=_NUM_CORES, num_subcores=_NUM_SUBCORES,
    )
    out = pl.kernel(
        _se3_body,
        out_type=jax.ShapeDtypeStruct((3, m * b, n), jnp.float32),
        mesh=mesh,
        scratch_types=(
            [
                [[pltpu.VMEM((_CHUNK,), jnp.float32) for _ in range(3)]
                 for _ in range(2)],
                [[pltpu.VMEM((_CHUNK,), jnp.float32) for _ in range(3 * _M)]
                 for _ in range(2)],
                [pltpu.VMEM((16,), jnp.float32) for _ in range(_M)],
                pltpu.SemaphoreType.DMA,
                [pltpu.SemaphoreType.DMA for _ in range(2)],
                [pltpu.SemaphoreType.DMA for _ in range(2)],
            ]
        ),
        compiler_params=pltpu.CompilerParams(needs_layout_passes=False),
    )(xyz, coef)
    return jnp.transpose(out, (1, 2, 0))


# speedup vs baseline: 1.0224x; 1.0224x over previous
import functools

import jax
import jax.numpy as jnp
from jax import lax
from jax.experimental import pallas as pl
from jax.experimental.pallas import tpu as pltpu
from jax.experimental.pallas import tpu_sc as plsc

_NUM_CORES = 2
_NUM_SUBCORES = 16
_LANES = 16

_M = 4
_B = 32
_N = 8192
_CHUNK = 2048
_NCHUNKS = _N // _CHUNK
_STEPS = _CHUNK // _LANES
_HALF = _N // 2


def _se3_body(xyz_hbm, coef_hbm, out_hbm, in_v, out_v, coef_v,
              coef_sem, in_sem, out_sem):
    c = lax.axis_index("c")
    s = lax.axis_index("s")
    w = s * _NUM_CORES + c

    for m in range(_M):
        pltpu.make_async_copy(
            coef_hbm.at[m * _B + w], coef_v[m], coef_sem).start()
    for h in range(2):
        for d in range(3):
            pltpu.make_async_copy(
                xyz_hbm.at[d, w, pl.ds(h * _HALF, _HALF)],
                in_v[d].at[pl.ds(h * _HALF, _HALF)], in_sem).start()
    for m in range(_M):
        pltpu.make_async_copy(
            coef_hbm.at[m * _B + w], coef_v[m], coef_sem).wait()
    rows = [coef_v[m][...] for m in range(_M)]
    cf = [[rows[m][j] for j in (0, 1, 2, 4, 5, 6, 8, 9, 10, 3, 7, 11)]
          for m in range(_M)]

    def half_wait(h):
        for d in range(3):
            pltpu.make_async_copy(
                xyz_hbm.at[d, w, pl.ds(h * _HALF, _HALF)],
                in_v[d].at[pl.ds(h * _HALF, _HALF)], in_sem).wait()

    def out_copy(k, j):
        d, row = j % 3, (j // 3) * _B + w
        return pltpu.make_async_copy(
            out_v[j].at[pl.ds(k * _CHUNK, _CHUNK)],
            out_hbm.at[d, row, pl.ds(k * _CHUNK, _CHUNK)],
            out_sem)

    xs, ys, zs = in_v

    def compute(k):
        base = k * _CHUNK

        @plsc.parallel_loop(0, _STEPS, unroll=4)
        def body(i):
            sl = pl.ds(base + i * _LANES, _LANES)
            x = xs[sl]
            y = ys[sl]
            z = zs[sl]
            for m in range(_M):
                r = cf[m]
                out_v[3 * m][sl] = x * r[0] + y * r[1] + z * r[2] + r[9]
                out_v[3 * m + 1][sl] = x * r[3] + y * r[4] + z * r[5] + r[10]
                out_v[3 * m + 2][sl] = x * r[6] + y * r[7] + z * r[8] + r[11]

    for k in range(_NCHUNKS):
        if k == 0:
            half_wait(0)
        if k == _NCHUNKS // 2:
            half_wait(1)
        compute(k)
        for j in range(3 * _M):
            out_copy(k, j).start()
    for k in range(_NCHUNKS):
        for j in range(3 * _M):
            out_copy(k, j).wait()


@jax.jit
def kernel(trans, pos):
    m, b = trans.shape[0], trans.shape[1]
    n = pos.shape[1]
    coef = trans.reshape(m * b, 16)
    xyz = jnp.transpose(pos, (2, 0, 1))

    mesh = plsc.VectorSubcoreMesh(
        core_axis_name="c", subcore_axis_name="s",
        num_cores=_NUM_CORES, num_subcores=_NUM_SUBCORES,
    )
    out = pl.kernel(
        _se3_body,
        out_type=jax.ShapeDtypeStruct((3, m * b, n), jnp.float32),
        mesh=mesh,
        scratch_types=(
            [
                [pltpu.VMEM((_N,), jnp.float32) for _ in range(3)],
                [pltpu.VMEM((_N,), jnp.float32) for _ in range(3 * _M)],
                [pltpu.VMEM((16,), jnp.float32) for _ in range(_M)],
                pltpu.SemaphoreType.DMA,
                pltpu.SemaphoreType.DMA,
                pltpu.SemaphoreType.DMA,
            ]
        ),
        compiler_params=pltpu.CompilerParams(needs_layout_passes=False),
    )(xyz, coef)
    return jnp.transpose(out, (1, 2, 0))


# speedup vs baseline: 1.2123x; 1.1858x over previous
import functools

import jax
import jax.numpy as jnp
from jax import lax
from jax.experimental import pallas as pl
from jax.experimental.pallas import tpu as pltpu
from jax.experimental.pallas import tpu_sc as plsc

_NUM_CORES = 2
_NUM_SUBCORES = 16
_LANES = 16

_M = 4
_B = 32
_N = 8192
_CHUNK = 2048
_NCHUNKS = _N // _CHUNK
_STEPS = _CHUNK // _LANES
_HALF = _N // 2


def _se3_body(xyz_hbm, coef_hbm, out_hbm, in_v, out_v, coef_v,
              coef_sem, in_sem, out_sem):
    c = lax.axis_index("c")
    s = lax.axis_index("s")
    w = s * _NUM_CORES + c

    for m in range(_M):
        pltpu.make_async_copy(
            coef_hbm.at[m * _B + w], coef_v[m], coef_sem).start()
    for h in range(2):
        for d in range(3):
            pltpu.make_async_copy(
                xyz_hbm.at[d, w, pl.ds(h * _HALF, _HALF)],
                in_v[d].at[pl.ds(h * _HALF, _HALF)], in_sem).start()
    for m in range(_M):
        pltpu.make_async_copy(
            coef_hbm.at[m * _B + w], coef_v[m], coef_sem).wait()
    rows = [coef_v[m][...] for m in range(_M)]
    cf = [[rows[m][j] for j in (0, 1, 2, 4, 5, 6, 8, 9, 10, 3, 7, 11)]
          for m in range(_M)]

    def half_wait(h):
        for d in range(3):
            pltpu.make_async_copy(
                xyz_hbm.at[d, w, pl.ds(h * _HALF, _HALF)],
                in_v[d].at[pl.ds(h * _HALF, _HALF)], in_sem).wait()

    def out_copy(k, j):
        d, row = j % 3, (j // 3) * _B + w
        return pltpu.make_async_copy(
            out_v[j].at[pl.ds(k * _CHUNK, _CHUNK)],
            out_hbm.at[d, row, pl.ds(k * _CHUNK, _CHUNK)],
            out_sem)

    xs, ys, zs = in_v

    def compute(k):
        base = k * _CHUNK

        @plsc.parallel_loop(0, _STEPS, unroll=2)
        def body(i):
            sl = pl.ds(base + i * _LANES, _LANES)
            x = xs[sl]
            y = ys[sl]
            z = zs[sl]
            for m in range(_M):
                r = cf[m]
                out_v[3 * m][sl] = x * r[0] + y * r[1] + z * r[2] + r[9]
                out_v[3 * m + 1][sl] = x * r[3] + y * r[4] + z * r[5] + r[10]
                out_v[3 * m + 2][sl] = x * r[6] + y * r[7] + z * r[8] + r[11]

    half_wait(0)

    def chunk_body(k, carry):
        @pl.when(k == _NCHUNKS // 2)
        def _():
            half_wait(1)
        compute(k)
        for j in range(3 * _M):
            out_copy(k, j).start()
        return carry

    lax.fori_loop(0, _NCHUNKS, chunk_body, 0)

    def drain_body(k, carry):
        for j in range(3 * _M):
            out_copy(k, j).wait()
        return carry

    lax.fori_loop(0, _NCHUNKS, drain_body, 0)


@jax.jit
def kernel(trans, pos):
    m, b = trans.shape[0], trans.shape[1]
    n = pos.shape[1]
    coef = trans.reshape(m * b, 16)
    xyz = jnp.transpose(pos, (2, 0, 1))

    mesh = plsc.VectorSubcoreMesh(
        core_axis_name="c", subcore_axis_name="s",
        num_cores=_NUM_CORES, num_subcores=_NUM_SUBCORES,
    )
    out = pl.kernel(
        _se3_body,
        out_type=jax.ShapeDtypeStruct((3, m * b, n), jnp.float32),
        mesh=mesh,
        scratch_types=(
            [
                [pltpu.VMEM((_N,), jnp.float32) for _ in range(3)],
                [pltpu.VMEM((_N,), jnp.float32) for _ in range(3 * _M)],
                [pltpu.VMEM((16,), jnp.float32) for _ in range(_M)],
                pltpu.SemaphoreType.DMA,
                pltpu.SemaphoreType.DMA,
                pltpu.SemaphoreType.DMA,
            ]
        ),
        compiler_params=pltpu.CompilerParams(needs_layout_passes=False),
    )(xyz, coef)
    return jnp.transpose(out, (1, 2, 0))
